# Initial kernel scaffold; baseline (speedup 1.0000x reference)
import jax
import jax.numpy as jnp
from jax.experimental import pallas as pl


def kernel(x, position, value_table, am):
    raise NotImplementedError("write your pallas kernel here")


# speedup vs baseline: 4.4759x; 4.4759x over previous
import jax
import jax.numpy as jnp
from jax.experimental import pallas as pl
from jax.experimental.pallas import tpu as pltpu

DIM = 2048
IMG = 784
LEVELS = 256
NUM_CLASSES = 10
BATCH = 128


def _hdc_body(x_ref, pos_ref, vt_ref, am_ref, out_ref):
    xrow = x_ref[0]
    idx = jnp.clip(jnp.round(xrow * (LEVELS - 1)), 0.0, LEVELS - 1.0).astype(jnp.int32)
    lvl = jax.lax.broadcasted_iota(jnp.int32, (LEVELS, IMG), 0)
    onehot = (lvl == idx).astype(jnp.bfloat16)
    pos = pos_ref[...]
    h = jax.lax.dot_general(
        onehot, pos, (((1,), (0,)), ((), ())),
        preferred_element_type=jnp.float32)
    enc = jnp.sum(h * vt_ref[...], axis=0, keepdims=True)
    am = am_ref[...]
    dots = jax.lax.dot_general(
        enc, am, (((1,), (1,)), ((), ())),
        preferred_element_type=jnp.float32)
    ne = jnp.sqrt(jnp.sum(enc * enc)) + 1e-12
    na = jnp.sqrt(jnp.sum(am * am, axis=1)).reshape(1, NUM_CLASSES) + 1e-12
    out_ref[0] = dots / ne / na


def kernel(x, position, value_table, am):
    flat = x.reshape(BATCH, 1, IMG)
    pos_bf = position.astype(jnp.bfloat16)
    out = pl.pallas_call(
        _hdc_body,
        grid=(BATCH,),
        in_specs=[
            pl.BlockSpec((1, 1, IMG), lambda i: (i, 0, 0)),
            pl.BlockSpec((IMG, DIM), lambda i: (0, 0)),
            pl.BlockSpec((LEVELS, DIM), lambda i: (0, 0)),
            pl.BlockSpec((NUM_CLASSES, DIM), lambda i: (0, 0)),
        ],
        out_specs=pl.BlockSpec((1, 1, NUM_CLASSES), lambda i: (i, 0, 0)),
        out_shape=jax.ShapeDtypeStruct((BATCH, 1, NUM_CLASSES), jnp.float32),
    )(flat, pos_bf, value_table, am)
    return out.reshape(BATCH, NUM_CLASSES)
